# baseline (speedup 1.0000x reference)
import functools

import jax
import jax.numpy as jnp
from jax import lax
from jax.experimental import pallas as pl
from jax.experimental.pallas import tpu as pltpu
from jax.experimental.pallas import tpu_sc as plsc

_SDF_MIN = -7.0
_SDF_MAX = 7.0
_N_BINS = 256
_K = (_N_BINS - 1) / (_SDF_MAX - _SDF_MIN)
_B = -_SDF_MIN * (_N_BINS - 1) / (_SDF_MAX - _SDF_MIN)
_MAGIC = 12582912.0
_MAGIC_BITS = 0x4B400000

_NC = 2
_NS = 16
_NW = _NC * _NS
_LANES = 16

_N_TOTAL = 8 * 128 * 128 * 128
_N_PER = _N_TOTAL // _NW
_CHUNK = 16384
_NCH = _N_PER // _CHUNK
_GRP = 4
_GROUPS = _CHUNK // (_LANES * _GRP)
_NBIN_FLAT = _LANES * _N_BINS


def _chunk_body(yp_buf, yt_buf, bins_v, lane_off):
    def body(i, c):
        base = i * (_LANES * _GRP)
        for j in range(_GRP):
            s = pl.ds(base + j * _LANES, _LANES)
            yt_v = yt_buf[s]
            yp_v = yp_buf[s]
            u = jnp.minimum(jnp.maximum(yt_v * _K + _B, 0.0),
                            float(_N_BINS - 1))
            t = lax.bitcast_convert_type(u + _MAGIC, jnp.int32)
            flat = t + lane_off
            d = yp_v - yt_v
            plsc.addupdate_scatter(bins_v, [flat], d * d)
        return c

    lax.fori_loop(0, _GROUPS, body, 0, unroll=2)


_mesh = plsc.VectorSubcoreMesh(core_axis_name="c", subcore_axis_name="s")


@functools.partial(
    pl.kernel,
    mesh=_mesh,
    compiler_params=pltpu.CompilerParams(needs_layout_passes=False),
    out_type=jax.ShapeDtypeStruct((_NW, _LANES), jnp.float32),
    scratch_types=[
        pltpu.VMEM((_N_BINS,), jnp.float32),
        pltpu.VMEM((_NBIN_FLAT,), jnp.float32),
        pltpu.VMEM((_CHUNK,), jnp.float32),
        pltpu.VMEM((_CHUNK,), jnp.float32),
        pltpu.VMEM((_CHUNK,), jnp.float32),
        pltpu.VMEM((_CHUNK,), jnp.float32),
        pltpu.SemaphoreType.DMA,
        pltpu.SemaphoreType.DMA,
        pltpu.VMEM((_LANES,), jnp.float32),
    ],
)
def _sc_loss(yp_hbm, yt_hbm, lut_hbm, out_hbm,
             lut_v, bins_v, yp_a, yp_b, yt_a, yt_b, sem_a, sem_b, acc_v):
    wid = lax.axis_index("s") * _NC + lax.axis_index("c")
    base = wid * _N_PER

    pltpu.sync_copy(lut_hbm, lut_v)

    lane_off = lax.iota(jnp.int32, _LANES) * _N_BINS - _MAGIC_BITS

    zero = jnp.zeros((_LANES,), jnp.float32)

    def clear(i, c):
        bins_v[pl.ds(i * _LANES, _LANES)] = zero
        return c

    lax.fori_loop(0, _NBIN_FLAT // _LANES, clear, 0)

    def start(buf_yp, buf_yt, sem, chunk_i):
        off = base + chunk_i * _CHUNK
        pltpu.make_async_copy(yp_hbm.at[pl.ds(off, _CHUNK)], buf_yp, sem).start()
        pltpu.make_async_copy(yt_hbm.at[pl.ds(off, _CHUNK)], buf_yt, sem).start()

    def wait(buf_yp, buf_yt, sem):
        pltpu.make_async_copy(yp_hbm.at[pl.ds(0, _CHUNK)], buf_yp, sem).wait()
        pltpu.make_async_copy(yt_hbm.at[pl.ds(0, _CHUNK)], buf_yt, sem).wait()

    start(yp_a, yt_a, sem_a, 0)
    start(yp_b, yt_b, sem_b, 1)

    def outer(g, c):
        wait(yp_a, yt_a, sem_a)
        _chunk_body(yp_a, yt_a, bins_v, lane_off)

        @pl.when(g < _NCH // 2 - 1)
        def _():
            start(yp_a, yt_a, sem_a, 2 * g + 2)

        wait(yp_b, yt_b, sem_b)
        _chunk_body(yp_b, yt_b, bins_v, lane_off)

        @pl.when(g < _NCH // 2 - 1)
        def _():
            start(yp_b, yt_b, sem_b, 2 * g + 3)

        return c

    lax.fori_loop(0, _NCH // 2, outer, 0)

    def reduce_body(i, acc):
        w = lut_v[pl.ds((i % 16) * _LANES, _LANES)]
        return acc + bins_v[pl.ds(i * _LANES, _LANES)] * w

    acc = lax.fori_loop(0, _NBIN_FLAT // _LANES, reduce_body, zero, unroll=4)
    acc_v[...] = acc
    pltpu.sync_copy(acc_v, out_hbm.at[wid])


def kernel(y_pred, y_true, lut):
    partials = _sc_loss(y_pred.reshape(-1), y_true.reshape(-1), lut)
    return partials.sum() / y_pred.size


# speedup vs baseline: 4.0736x; 4.0736x over previous
import functools

import jax
import jax.numpy as jnp
from jax import lax
from jax.experimental import pallas as pl
from jax.experimental.pallas import tpu as pltpu
from jax.experimental.pallas import tpu_sc as plsc

_SDF_MIN = -7.0
_SDF_MAX = 7.0
_N_BINS = 256
_K = (_N_BINS - 1) / (_SDF_MAX - _SDF_MIN)
_B = -_SDF_MIN * (_N_BINS - 1) / (_SDF_MAX - _SDF_MIN)
_MAGIC = 12582912.0

_NC = 2
_NS = 16
_NW = _NC * _NS
_LANES = 16

_N_TOTAL = 8 * 128 * 128 * 128
_N_PER = _N_TOTAL // _NW
_CHUNK = 16384
_NCH = _N_PER // _CHUNK
_NACC = 4
_GROUPS = _CHUNK // (_LANES * _NACC)


def _chunk_body(yp_buf, yt_buf, lut_v, accs):
    def body(i, accs):
        base = i * (_LANES * _NACC)
        out = []
        for j, acc in enumerate(accs):
            s = pl.ds(base + j * _LANES, _LANES)
            yt_v = yt_buf[s]
            yp_v = yp_buf[s]
            u = jnp.minimum(jnp.maximum(yt_v * _K + (_B + 0.5), 0.5),
                            _N_BINS - 0.5)
            idx = u.astype(jnp.int32)
            w = plsc.load_gather(lut_v, [idx])
            d = yp_v - yt_v
            out.append(acc + w * (d * d))
        return tuple(out)

    return lax.fori_loop(0, _GROUPS, body, accs, unroll=2)


_mesh = plsc.VectorSubcoreMesh(core_axis_name="c", subcore_axis_name="s")


@functools.partial(
    pl.kernel,
    mesh=_mesh,
    compiler_params=pltpu.CompilerParams(needs_layout_passes=False),
    out_type=jax.ShapeDtypeStruct((_NW, _LANES), jnp.float32),
    scratch_types=[
        pltpu.VMEM((_N_BINS,), jnp.float32),
        pltpu.VMEM((_CHUNK,), jnp.float32),
        pltpu.VMEM((_CHUNK,), jnp.float32),
        pltpu.VMEM((_CHUNK,), jnp.float32),
        pltpu.VMEM((_CHUNK,), jnp.float32),
        pltpu.SemaphoreType.DMA,
        pltpu.SemaphoreType.DMA,
        pltpu.VMEM((_LANES,), jnp.float32),
    ],
)
def _sc_loss(yp_hbm, yt_hbm, lut_hbm, out_hbm,
             lut_v, yp_a, yp_b, yt_a, yt_b, sem_a, sem_b, acc_v):
    wid = lax.axis_index("s") * _NC + lax.axis_index("c")
    base = wid * _N_PER

    pltpu.sync_copy(lut_hbm, lut_v)

    def start(buf_yp, buf_yt, sem, chunk_i):
        off = base + chunk_i * _CHUNK
        pltpu.make_async_copy(yp_hbm.at[pl.ds(off, _CHUNK)], buf_yp, sem).start()
        pltpu.make_async_copy(yt_hbm.at[pl.ds(off, _CHUNK)], buf_yt, sem).start()

    def wait(buf_yp, buf_yt, sem):
        pltpu.make_async_copy(yp_hbm.at[pl.ds(0, _CHUNK)], buf_yp, sem).wait()
        pltpu.make_async_copy(yt_hbm.at[pl.ds(0, _CHUNK)], buf_yt, sem).wait()

    start(yp_a, yt_a, sem_a, 0)
    start(yp_b, yt_b, sem_b, 1)

    def outer(g, accs):
        wait(yp_a, yt_a, sem_a)
        accs = _chunk_body(yp_a, yt_a, lut_v, accs)

        @pl.when(g < _NCH // 2 - 1)
        def _():
            start(yp_a, yt_a, sem_a, 2 * g + 2)

        wait(yp_b, yt_b, sem_b)
        accs = _chunk_body(yp_b, yt_b, lut_v, accs)

        @pl.when(g < _NCH // 2 - 1)
        def _():
            start(yp_b, yt_b, sem_b, 2 * g + 3)

        return accs

    zero = jnp.zeros((_LANES,), jnp.float32)
    accs = lax.fori_loop(0, _NCH // 2, outer, (zero,) * _NACC)
    acc = (accs[0] + accs[1]) + (accs[2] + accs[3])
    acc_v[...] = acc
    pltpu.sync_copy(acc_v, out_hbm.at[wid])


def kernel(y_pred, y_true, lut):
    partials = _sc_loss(y_pred.reshape(-1), y_true.reshape(-1), lut)
    return partials.sum() / y_pred.size


# speedup vs baseline: 4.2080x; 1.0330x over previous
import functools

import jax
import jax.numpy as jnp
from jax import lax
from jax.experimental import pallas as pl
from jax.experimental.pallas import tpu as pltpu
from jax.experimental.pallas import tpu_sc as plsc

_SDF_MIN = -7.0
_SDF_MAX = 7.0
_N_BINS = 256
_K = (_N_BINS - 1) / (_SDF_MAX - _SDF_MIN)
_B = -_SDF_MIN * (_N_BINS - 1) / (_SDF_MAX - _SDF_MIN)
_MAGIC = 12582912.0

_NC = 2
_NS = 16
_NW = _NC * _NS
_LANES = 16

_N_TOTAL = 8 * 128 * 128 * 128
_N_PER = _N_TOTAL // _NW
_CHUNK = 16384
_NCH = _N_PER // _CHUNK
_NACC = 4
_GROUPS = _CHUNK // (_LANES * _NACC)


def _chunk_body(yp_buf, yt_buf, lut_v, accs):
    def body(i, accs):
        base = i * (_LANES * _NACC)
        out = []
        for j, acc in enumerate(accs):
            s = pl.ds(base + j * _LANES, _LANES)
            yt_v = yt_buf[s]
            yp_v = yp_buf[s]
            u = jnp.minimum(jnp.maximum(yt_v * _K + _B, 0.0), float(_N_BINS - 1))
            t = lax.bitcast_convert_type(u + _MAGIC, jnp.int32)
            idx = jnp.bitwise_and(t, _N_BINS - 1)
            w = plsc.load_gather(lut_v, [idx])
            d = yp_v - yt_v
            out.append(acc + w * (d * d))
        return tuple(out)

    return lax.fori_loop(0, _GROUPS, body, accs, unroll=4)


_mesh = plsc.VectorSubcoreMesh(core_axis_name="c", subcore_axis_name="s")


@functools.partial(
    pl.kernel,
    mesh=_mesh,
    compiler_params=pltpu.CompilerParams(needs_layout_passes=False),
    out_type=jax.ShapeDtypeStruct((_NW, _LANES), jnp.float32),
    scratch_types=[
        pltpu.VMEM((_N_BINS,), jnp.float32),
        pltpu.VMEM((_CHUNK,), jnp.float32),
        pltpu.VMEM((_CHUNK,), jnp.float32),
        pltpu.VMEM((_CHUNK,), jnp.float32),
        pltpu.VMEM((_CHUNK,), jnp.float32),
        pltpu.SemaphoreType.DMA,
        pltpu.SemaphoreType.DMA,
        pltpu.VMEM((_LANES,), jnp.float32),
    ],
)
def _sc_loss(yp_hbm, yt_hbm, lut_hbm, out_hbm,
             lut_v, yp_a, yp_b, yt_a, yt_b, sem_a, sem_b, acc_v):
    wid = lax.axis_index("s") * _NC + lax.axis_index("c")
    base = wid * _N_PER

    pltpu.sync_copy(lut_hbm, lut_v)

    def start(buf_yp, buf_yt, sem, chunk_i):
        off = base + chunk_i * _CHUNK
        pltpu.make_async_copy(yp_hbm.at[pl.ds(off, _CHUNK)], buf_yp, sem).start()
        pltpu.make_async_copy(yt_hbm.at[pl.ds(off, _CHUNK)], buf_yt, sem).start()

    def wait(buf_yp, buf_yt, sem):
        pltpu.make_async_copy(yp_hbm.at[pl.ds(0, _CHUNK)], buf_yp, sem).wait()
        pltpu.make_async_copy(yt_hbm.at[pl.ds(0, _CHUNK)], buf_yt, sem).wait()

    start(yp_a, yt_a, sem_a, 0)
    start(yp_b, yt_b, sem_b, 1)

    def outer(g, accs):
        wait(yp_a, yt_a, sem_a)
        accs = _chunk_body(yp_a, yt_a, lut_v, accs)

        @pl.when(g < _NCH // 2 - 1)
        def _():
            start(yp_a, yt_a, sem_a, 2 * g + 2)

        wait(yp_b, yt_b, sem_b)
        accs = _chunk_body(yp_b, yt_b, lut_v, accs)

        @pl.when(g < _NCH // 2 - 1)
        def _():
            start(yp_b, yt_b, sem_b, 2 * g + 3)

        return accs

    zero = jnp.zeros((_LANES,), jnp.float32)
    accs = lax.fori_loop(0, _NCH // 2, outer, (zero,) * _NACC)
    acc = (accs[0] + accs[1]) + (accs[2] + accs[3])
    acc_v[...] = acc
    pltpu.sync_copy(acc_v, out_hbm.at[wid])


def kernel(y_pred, y_true, lut):
    partials = _sc_loss(y_pred.reshape(-1), y_true.reshape(-1), lut)
    return partials.sum() / y_pred.size
